# baseline (speedup 1.0000x reference)
import functools

import jax
import jax.numpy as jnp
from jax import lax
from jax.experimental import pallas as pl
from jax.experimental.pallas import tpu as pltpu

_RATIO = 0.7
_H = 512
_W = 512


def _bbox_body(mask_ref, bbox_ref):
    m = mask_ref[...]
    nz = m != 0.0
    row_id = lax.broadcasted_iota(jnp.int32, (_H, _W), 0)
    col_id = lax.broadcasted_iota(jnp.int32, (_H, _W), 1)
    top = jnp.min(jnp.where(nz, row_id, _H))
    bottom = jnp.max(jnp.where(nz, row_id, -1))
    left = jnp.min(jnp.where(nz, col_id, _W))
    right = jnp.max(jnp.where(nz, col_id, -1))
    bbox_ref[0] = jnp.floor(top * _RATIO).astype(jnp.int32)
    bbox_ref[1] = jnp.floor(left * _RATIO).astype(jnp.int32)
    bbox_ref[2] = jnp.floor(bottom + (_H - bottom) * (1.0 - _RATIO)).astype(jnp.int32)
    bbox_ref[3] = jnp.floor(right + (_W - right) * (1.0 - _RATIO)).astype(jnp.int32)


def _crop_body(bbox_ref, img_ref, out_ref):
    t = bbox_ref[0]
    l = bbox_ref[1]
    b = bbox_ref[2]
    r = bbox_ref[3]
    row_id = lax.broadcasted_iota(jnp.int32, (_H, _W), 0)
    col_id = lax.broadcasted_iota(jnp.int32, (_H, _W), 1)
    region = (row_id >= t) & (row_id < b) & (col_id >= l) & (col_id < r)
    out_ref[...] = jnp.where(region[None, :, :], img_ref[...], -1.0)


@jax.jit
def kernel(image, cover, mask):
    del cover
    m = mask[0, 0]
    bbox = pl.pallas_call(
        _bbox_body,
        out_shape=jax.ShapeDtypeStruct((4,), jnp.int32),
        in_specs=[pl.BlockSpec(memory_space=pltpu.VMEM)],
        out_specs=pl.BlockSpec(memory_space=pltpu.SMEM),
    )(m)

    n = image.shape[0] * image.shape[1]
    x = image.reshape(n, _H, _W)
    blk = 4
    grid_spec = pltpu.PrefetchScalarGridSpec(
        num_scalar_prefetch=1,
        grid=(n // blk,),
        in_specs=[pl.BlockSpec((blk, _H, _W), lambda i, bbox: (i, 0, 0))],
        out_specs=pl.BlockSpec((blk, _H, _W), lambda i, bbox: (i, 0, 0)),
    )
    out = pl.pallas_call(
        _crop_body,
        grid_spec=grid_spec,
        out_shape=jax.ShapeDtypeStruct((n, _H, _W), jnp.float32),
        compiler_params=pltpu.CompilerParams(
            dimension_semantics=("parallel",),
        ),
    )(bbox, x)
    return out.reshape(image.shape)


# speedup vs baseline: 1.0451x; 1.0451x over previous
import functools

import jax
import jax.numpy as jnp
from jax import lax
from jax.experimental import pallas as pl
from jax.experimental.pallas import tpu as pltpu

_RATIO = 0.7
_H = 512
_W = 512


def _bbox_body(mask_ref, bbox_ref):
    m = mask_ref[...]
    nz = m != 0.0
    row_id = lax.broadcasted_iota(jnp.int32, (_H, _W), 0)
    col_id = lax.broadcasted_iota(jnp.int32, (_H, _W), 1)
    top = jnp.min(jnp.where(nz, row_id, _H))
    bottom = jnp.max(jnp.where(nz, row_id, -1))
    left = jnp.min(jnp.where(nz, col_id, _W))
    right = jnp.max(jnp.where(nz, col_id, -1))
    bbox_ref[0] = jnp.floor(top * _RATIO).astype(jnp.int32)
    bbox_ref[1] = jnp.floor(left * _RATIO).astype(jnp.int32)
    bbox_ref[2] = jnp.floor(bottom + (_H - bottom) * (1.0 - _RATIO)).astype(jnp.int32)
    bbox_ref[3] = jnp.floor(right + (_W - right) * (1.0 - _RATIO)).astype(jnp.int32)


def _crop_body(bbox_ref, img_ref, out_ref):
    t = bbox_ref[0]
    l = bbox_ref[1]
    b = bbox_ref[2]
    r = bbox_ref[3]
    row_id = lax.broadcasted_iota(jnp.int32, (_H, _W), 0)
    col_id = lax.broadcasted_iota(jnp.int32, (_H, _W), 1)
    region = (row_id >= t) & (row_id < b) & (col_id >= l) & (col_id < r)
    out_ref[...] = jnp.where(region[None, :, :], img_ref[...], -1.0)


@jax.jit
def kernel(image, cover, mask):
    del cover
    m = mask[0, 0]
    bbox = pl.pallas_call(
        _bbox_body,
        out_shape=jax.ShapeDtypeStruct((4,), jnp.int32),
        in_specs=[pl.BlockSpec(memory_space=pltpu.VMEM)],
        out_specs=pl.BlockSpec(memory_space=pltpu.SMEM),
    )(m)

    n = image.shape[0] * image.shape[1]
    x = image.reshape(n, _H, _W)
    blk = 12
    grid_spec = pltpu.PrefetchScalarGridSpec(
        num_scalar_prefetch=1,
        grid=(n // blk,),
        in_specs=[pl.BlockSpec((blk, _H, _W), lambda i, bbox: (i, 0, 0))],
        out_specs=pl.BlockSpec((blk, _H, _W), lambda i, bbox: (i, 0, 0)),
    )
    out = pl.pallas_call(
        _crop_body,
        grid_spec=grid_spec,
        out_shape=jax.ShapeDtypeStruct((n, _H, _W), jnp.float32),
        compiler_params=pltpu.CompilerParams(
            dimension_semantics=("parallel",),
        ),
    )(bbox, x)
    return out.reshape(image.shape)
